# baseline (speedup 1.0000x reference)
import functools
import jax
import jax.numpy as jnp
from jax import lax
from jax.experimental import pallas as pl
from jax.experimental.pallas import tpu as pltpu
from jax.experimental.pallas import tpu_sc as plsc

D_MODEL = 64
SCALE = 8.0
LANES = 16

_NC = 2
_NS = 16
_NW = _NC * _NS

_B = 4096 * 200
_BPW = _B // _NW
_CHUNK = 128
_NCHUNK = _BPW // _CHUNK
_NBUF = 4
_AHEAD = 2

_VROWS = 1000000 // 2
_OROWS = _B // 2
_OPC = _CHUNK // 2


@functools.cache
def _build_embed_sc():
    mesh = plsc.VectorSubcoreMesh(core_axis_name="c", subcore_axis_name="s")

    @functools.partial(
        pl.kernel,
        mesh=mesh,
        compiler_params=pltpu.CompilerParams(needs_layout_passes=False),
        out_type=jax.ShapeDtypeStruct((_OROWS, 2 * D_MODEL), jnp.float32),
        scratch_types=[
            pltpu.VMEM((_NCHUNK, _CHUNK), jnp.int32),
            pltpu.VMEM((_NBUF, _CHUNK), jnp.int32),
            pltpu.VMEM((_NBUF, _CHUNK, 2 * D_MODEL), jnp.float32),
            pltpu.VMEM((_NBUF, _OPC, 2 * D_MODEL), jnp.float32),
            [pltpu.SemaphoreType.DMA] * _NBUF,
            [pltpu.SemaphoreType.DMA] * _NBUF,
        ],
    )
    def _embed_sc(idx_hbm, table_hbm, out_hbm, idx_v, pidx_v, gbuf, wbuf,
                  gsems, wsems):
        wid = lax.axis_index("s") * _NC + lax.axis_index("c")
        base2 = wid * (_BPW // 2)
        pltpu.sync_copy(idx_hbm.at[wid], idx_v)

        def start_gather(g, b):
            for k in range(_CHUNK // LANES):
                sl = pl.ds(k * LANES, LANES)
                pidx_v[b, sl] = idx_v[g, sl] >> 1
            pltpu.async_copy(table_hbm.at[pidx_v.at[b]], gbuf.at[b], gsems[b])

        def wait_gather(b):
            pltpu.make_async_copy(
                out_hbm.at[pl.ds(0, _CHUNK)], gbuf.at[b], gsems[b]
            ).wait()

        def start_write(g, b):
            pltpu.async_copy(
                wbuf.at[b], out_hbm.at[pl.ds(base2 + g * _OPC, _OPC)],
                wsems[b],
            )

        def wait_write(b):
            pltpu.make_async_copy(
                wbuf.at[b], out_hbm.at[pl.ds(0, _OPC)], wsems[b]
            ).wait()

        def scale(g, b):
            iota = lax.iota(jnp.int32, LANES)
            for t in range(_CHUNK // LANES):
                idxv = idx_v[g, pl.ds(t * LANES, LANES)]
                hoff = (idxv & 1) * D_MODEL
                srow = t * LANES + iota
                qrow = srow >> 1
                pcol = (iota & 1) * D_MODEL

                def cbody(c, _hoff=hoff, _srow=srow, _qrow=qrow, _pcol=pcol):
                    val = plsc.load_gather(gbuf.at[b], [_srow, _hoff + c])
                    plsc.store_scatter(wbuf.at[b], [_qrow, _pcol + c],
                                       val * SCALE)

                plsc.parallel_loop(0, D_MODEL, unroll=4)(cbody)

        start_gather(0, 0)
        start_gather(1, 1)

        for g in range(_NBUF):
            start_gather(g + _AHEAD, (g + _AHEAD) % _NBUF)
            wait_gather(g % _NBUF)
            scale(g, g % _NBUF)
            start_write(g, g % _NBUF)

        _NSTEADY = (_NCHUNK - _NBUF - _AHEAD) // _NBUF

        def block(kb, carry):
            for j in range(_NBUF):
                g = _NBUF + kb * _NBUF + j
                start_gather(g + _AHEAD, (j + _AHEAD) % _NBUF)
                wait_gather(j)
                wait_write(j)
                scale(g, j)
                start_write(g, j)
            return carry

        lax.fori_loop(0, _NSTEADY, block, 0)

        for g in range(_NBUF + _NSTEADY * _NBUF, _NCHUNK - _AHEAD):
            b = g % _NBUF
            start_gather(g + _AHEAD, (g + _AHEAD) % _NBUF)
            wait_gather(b)
            wait_write(b)
            scale(g, b)
            start_write(g, b)

        for g in range(_NCHUNK - _AHEAD, _NCHUNK):
            b = g % _NBUF
            wait_gather(b)
            wait_write(b)
            scale(g, b)
            start_write(g, b)

        for b in range(_NBUF):
            wait_write(b)

    return _embed_sc


def kernel(x, table):
    flat_idx = x.reshape(-1).astype(jnp.int32).reshape(_NW, _NCHUNK, _CHUNK)
    table2 = table.reshape(_VROWS, 2 * D_MODEL)
    out2 = _build_embed_sc()(flat_idx, table2)
    return out2.reshape(x.shape + (D_MODEL,))


# speedup vs baseline: 1.6496x; 1.6496x over previous
import functools
import jax
import jax.numpy as jnp
from jax import lax
from jax.experimental import pallas as pl
from jax.experimental.pallas import tpu as pltpu
from jax.experimental.pallas import tpu_sc as plsc

D_MODEL = 64
SCALE = 8.0
LANES = 16

_NC = 2
_NS = 16
_NW = _NC * _NS

_BATCH = 4096
_SEQ = 200
_BB = _BATCH // _NW
_NBUF = 4
_AHEAD = 2

_VROWS = 1000000 // 2


@functools.cache
def _build_embed_sc():
    mesh = plsc.VectorSubcoreMesh(core_axis_name="c", subcore_axis_name="s")

    @functools.partial(
        pl.kernel,
        mesh=mesh,
        compiler_params=pltpu.CompilerParams(needs_layout_passes=False),
        out_type=jax.ShapeDtypeStruct((_SEQ, D_MODEL, _BATCH), jnp.float32),
        scratch_types=[
            pltpu.VMEM((_SEQ, _BB), jnp.int32),
            pltpu.VMEM((_NBUF, _BB), jnp.int32),
            pltpu.VMEM((_NBUF, _BB, 2 * D_MODEL), jnp.float32),
            pltpu.VMEM((_NBUF, D_MODEL, _BB), jnp.float32),
            [pltpu.SemaphoreType.DMA] * _NBUF,
            [pltpu.SemaphoreType.DMA] * _NBUF,
        ],
    )
    def _embed_sc(xt_hbm, table_hbm, out_hbm, idx_v, pidx_v, gbuf, wbuf,
                  gsems, wsems):
        wid = lax.axis_index("s") * _NC + lax.axis_index("c")
        b0 = wid * _BB
        pltpu.sync_copy(xt_hbm.at[:, pl.ds(b0, _BB)], idx_v)

        def start_gather(s, b):
            for k in range(_BB // LANES):
                sl = pl.ds(k * LANES, LANES)
                pidx_v[b, sl] = idx_v[s, sl] >> 1
            pltpu.async_copy(table_hbm.at[pidx_v.at[b]], gbuf.at[b], gsems[b])

        def wait_gather(b):
            pltpu.make_async_copy(
                table_hbm.at[pl.ds(0, _BB)], gbuf.at[b], gsems[b]
            ).wait()

        def start_write(s, b):
            pltpu.async_copy(
                wbuf.at[b], out_hbm.at[s, :, pl.ds(b0, _BB)], wsems[b]
            )

        def wait_write(b):
            pltpu.make_async_copy(
                wbuf.at[b], out_hbm.at[0, :, pl.ds(0, _BB)], wsems[b]
            ).wait()

        iota = lax.iota(jnp.int32, LANES)

        def scale(s, b):
            for rb in range(_BB // LANES):
                idxv = idx_v[s, pl.ds(rb * LANES, LANES)]
                hoff = (idxv & 1) * D_MODEL
                srow = rb * LANES + iota

                def dbody(d, _hoff=hoff, _srow=srow, _rb=rb):
                    val = plsc.load_gather(gbuf.at[b], [_srow, _hoff + d])
                    wbuf[b, d, pl.ds(_rb * LANES, LANES)] = val * SCALE

                plsc.parallel_loop(0, D_MODEL, unroll=4)(dbody)

        start_gather(0, 0)
        start_gather(1, 1)

        for s in range(_NBUF):
            start_gather(s + _AHEAD, (s + _AHEAD) % _NBUF)
            wait_gather(s % _NBUF)
            scale(s, s % _NBUF)
            start_write(s, s % _NBUF)

        _NSTEADY = (_SEQ - _NBUF - _AHEAD) // _NBUF

        def block(kb, carry):
            for j in range(_NBUF):
                s = _NBUF + kb * _NBUF + j
                start_gather(s + _AHEAD, (j + _AHEAD) % _NBUF)
                wait_gather(j)
                wait_write(j)
                scale(s, j)
                start_write(s, j)
            return carry

        lax.fori_loop(0, _NSTEADY, block, 0)

        for s in range(_NBUF + _NSTEADY * _NBUF, _SEQ - _AHEAD):
            b = s % _NBUF
            start_gather(s + _AHEAD, (s + _AHEAD) % _NBUF)
            wait_gather(b)
            wait_write(b)
            scale(s, b)
            start_write(s, b)

        for s in range(_SEQ - _AHEAD, _SEQ):
            b = s % _NBUF
            wait_gather(b)
            wait_write(b)
            scale(s, b)
            start_write(s, b)

        for b in range(_NBUF):
            wait_write(b)

    return _embed_sc


def kernel(x, table):
    xt = x.T.astype(jnp.int32)
    table2 = table.reshape(_VROWS, 2 * D_MODEL)
    out = _build_embed_sc()(xt, table2)
    return jnp.permute_dims(out, (2, 0, 1))


# speedup vs baseline: 1.6832x; 1.0204x over previous
import functools
import jax
import jax.numpy as jnp
from jax import lax
from jax.experimental import pallas as pl
from jax.experimental.pallas import tpu as pltpu
from jax.experimental.pallas import tpu_sc as plsc

D_MODEL = 64
SCALE = 8.0
LANES = 16

_NC = 2
_NS = 16
_NW = _NC * _NS

_BATCH = 4096
_SEQ = 200
_BB = _BATCH // _NW
_NBUF = 4
_AHEAD = 2

_VOCAB = 1000000


@functools.cache
def _build_embed_sc():
    mesh = plsc.VectorSubcoreMesh(core_axis_name="c", subcore_axis_name="s")

    @functools.partial(
        pl.kernel,
        mesh=mesh,
        compiler_params=pltpu.CompilerParams(
            use_tc_tiling_on_sc=False, needs_layout_passes=False
        ),
        out_type=jax.ShapeDtypeStruct(
            (_SEQ, D_MODEL // 8, _BATCH // _BB, 8, _BB), jnp.float32
        ),
        scratch_types=[
            pltpu.VMEM((_SEQ, _BB), jnp.int32),
            pltpu.VMEM((_NBUF, _BB, D_MODEL), jnp.float32),
            pltpu.VMEM((_NBUF, D_MODEL // 8, 8, _BB), jnp.float32),
            [pltpu.SemaphoreType.DMA] * _NBUF,
            [pltpu.SemaphoreType.DMA] * _NBUF,
        ],
    )
    def _embed_sc(xt_hbm, table_hbm, out_hbm, idx_v, gbuf, wbuf, gsems, wsems):
        wid = lax.axis_index("s") * _NC + lax.axis_index("c")
        b0 = wid * _BB
        pltpu.sync_copy(xt_hbm.at[:, pl.ds(b0, _BB)], idx_v)

        def start_gather(s, b):
            pltpu.async_copy(table_hbm.at[idx_v.at[s]], gbuf.at[b], gsems[b])

        def wait_gather(b):
            pltpu.make_async_copy(
                table_hbm.at[pl.ds(0, _BB)], gbuf.at[b], gsems[b]
            ).wait()

        def start_write(s, b):
            pltpu.async_copy(
                wbuf.at[b], out_hbm.at[s, :, wid, :, :], wsems[b]
            )

        def wait_write(b):
            pltpu.make_async_copy(
                wbuf.at[b], out_hbm.at[0, :, 0, :, :], wsems[b]
            ).wait()

        iota = lax.iota(jnp.int32, LANES)

        def scale(s, b):
            for rb in range(_BB // LANES):
                srow = rb * LANES + iota

                def dbody(d, _srow=srow, _rb=rb):
                    col = jnp.full((LANES,), 0, jnp.int32) + d
                    val = plsc.load_gather(gbuf.at[b], [_srow, col])
                    wbuf[b, d >> 3, d & 7, pl.ds(_rb * LANES, LANES)] = (
                        val * SCALE
                    )

                plsc.parallel_loop(0, D_MODEL, unroll=4)(dbody)

        start_gather(0, 0)
        start_gather(1, 1)

        for s in range(_NBUF):
            start_gather(s + _AHEAD, (s + _AHEAD) % _NBUF)
            wait_gather(s % _NBUF)
            scale(s, s % _NBUF)
            start_write(s, s % _NBUF)

        _NSTEADY = (_SEQ - _NBUF - _AHEAD) // _NBUF

        def block(kb, carry):
            for j in range(_NBUF):
                s = _NBUF + kb * _NBUF + j
                start_gather(s + _AHEAD, (j + _AHEAD) % _NBUF)
                wait_gather(j)
                wait_write(j)
                scale(s, j)
                start_write(s, j)
            return carry

        lax.fori_loop(0, _NSTEADY, block, 0)

        for s in range(_NBUF + _NSTEADY * _NBUF, _SEQ - _AHEAD):
            b = s % _NBUF
            start_gather(s + _AHEAD, (s + _AHEAD) % _NBUF)
            wait_gather(b)
            wait_write(b)
            scale(s, b)
            start_write(s, b)

        for s in range(_SEQ - _AHEAD, _SEQ):
            b = s % _NBUF
            wait_gather(b)
            wait_write(b)
            scale(s, b)
            start_write(s, b)

        for b in range(_NBUF):
            wait_write(b)

    return _embed_sc


def kernel(x, table):
    xt = x.T.astype(jnp.int32)
    out5 = _build_embed_sc()(xt, table)
    out = jnp.permute_dims(out5, (2, 4, 0, 1, 3))
    return out.reshape(_BATCH, _SEQ, D_MODEL)
